# baseline (speedup 1.0000x reference)
import functools

import jax
import jax.numpy as jnp
from jax import lax
from jax.experimental import pallas as pl
from jax.experimental.pallas import tpu as pltpu
from jax.experimental.pallas import tpu_sc as plsc

BATCH = 4096
MAX_LEN = 200
EMBED = 64
LP = 208
HALF = LP // 2
VOCAB2 = 500000
NC, NS, LANES = 2, 16, 16
NW = NC * NS
BPW = BATCH // NW
UPW = 2 * BPW
NCH = EMBED // LANES
NB = 4
STW = 112


def _encoder(xf_hbm, t128_hbm, f_hbm, out_hbm,
             idx_v, stage_v, f_v, buf0, buf1, buf2, buf3, out_v,
             sem0, sem1, sem2, sem3):
    bufs = (buf0, buf1, buf2, buf3)
    sems = (sem0, sem1, sem2, sem3)
    wid = lax.axis_index("s") * NC + lax.axis_index("c")
    base = wid * BPW

    pltpu.sync_copy(xf_hbm.at[pl.ds(base * LP, BPW * LP)],
                    idx_v.at[pl.ds(0, BPW * LP)])
    pltpu.sync_copy(f_hbm, f_v)

    def fire(u, slot, buf, sem):
        pu = u * HALF
        for k in range(7):
            o = 16 * k if k < 6 else 96
            v = idx_v[pl.ds(pu + o, 16)]
            stage_v[slot, pl.ds(o, 16)] = jnp.right_shift(v, 1)
        pltpu.make_async_copy(
            t128_hbm.at[stage_v.at[slot, pl.ds(0, HALF)]], buf, sem).start()

    def drain(buf, sem):
        pltpu.make_async_copy(
            t128_hbm.at[stage_v.at[0, pl.ds(0, HALF)]], buf, sem).wait()

    def accumulate(u, j, buf, acc):
        fbase = (j & 1) * HALF
        pu = u * HALF

        def body(rr, acc):
            r0 = rr * 8
            iv_vec = idx_v[pl.ds(pu + r0, 16)]
            for i in range(8):
                r = r0 + i
                off = (iv_vec[i] & 1) * EMBED
                acc = tuple(
                    acc[c] + buf[r, pl.ds(off + c * LANES, LANES)]
                    * f_v[fbase + r, pl.ds(c * LANES, LANES)]
                    for c in range(NCH))
            return acc
        return lax.fori_loop(0, HALF // 8, body, acc)

    zeros = tuple(jnp.zeros((LANES,), jnp.float32) for _ in range(NCH))
    for u in range(NB - 1):
        fire(u, u, bufs[u], sems[u])

    def gbody(g, carry):
        acc = zeros
        for j in range(NB):
            u = NB * g + j
            drain(bufs[j], sems[j])
            acc = accumulate(u, j, bufs[j], acc)
            if j & 1:
                b = u // 2
                for c in range(NCH):
                    out_v[b, pl.ds(c * LANES, LANES)] = acc[c]
                acc = zeros
            jn = (j + NB - 1) % NB

            @pl.when(u + NB - 1 < UPW)
            def _():
                fire(u + NB - 1, jn, bufs[jn], sems[jn])

        return carry

    lax.fori_loop(0, UPW // NB, gbody, 0)

    pltpu.sync_copy(out_v, out_hbm.at[pl.ds(base, BPW)])


_mesh = plsc.VectorSubcoreMesh(core_axis_name="c", subcore_axis_name="s")

_enc = functools.partial(
    pl.kernel, mesh=_mesh,
    compiler_params=pltpu.CompilerParams(use_tc_tiling_on_sc=False),
    out_type=jax.ShapeDtypeStruct((BATCH, EMBED), jnp.float32),
    scratch_types=[
        pltpu.VMEM((BPW * LP + 16,), jnp.int32),
        pltpu.VMEM((NB, STW), jnp.int32),
        pltpu.VMEM((LP, EMBED), jnp.float32),
        pltpu.VMEM((HALF, 2 * EMBED), jnp.float32),
        pltpu.VMEM((HALF, 2 * EMBED), jnp.float32),
        pltpu.VMEM((HALF, 2 * EMBED), jnp.float32),
        pltpu.VMEM((HALF, 2 * EMBED), jnp.float32),
        pltpu.VMEM((BPW, EMBED), jnp.float32),
        pltpu.SemaphoreType.DMA,
        pltpu.SemaphoreType.DMA,
        pltpu.SemaphoreType.DMA,
        pltpu.SemaphoreType.DMA,
    ],
)(_encoder)


@jax.jit
def kernel(x, table, f):
    xp = jnp.pad(x.astype(jnp.int32), ((0, 0), (0, LP - MAX_LEN))).reshape(-1)
    fp = jnp.pad(f, ((0, LP - MAX_LEN), (0, 0)))
    t128 = table.reshape(VOCAB2, 2 * EMBED)
    return _enc(xp, t128, fp)
